# baseline (speedup 1.0000x reference)
import jax
import jax.numpy as jnp
from jax import lax
from jax.experimental import pallas as pl
from jax.experimental.pallas import tpu as pltpu
from jax.experimental.pallas import tpu_sc as plsc

NC, NS, L = 2, 16, 16
NW = NC * NS


def _mesh():
    return plsc.VectorSubcoreMesh(
        core_axis_name="c", subcore_axis_name="s", num_cores=NC, num_subcores=NS
    )


def _params():
    return pltpu.CompilerParams(needs_layout_passes=False)


def _worker_id():
    return lax.axis_index("s") * NC + lax.axis_index("c")


def _build_posof(idx_v, posof_v, B):
    lid = lax.iota(jnp.int32, L)

    @pl.loop(0, B // L)
    def _(c):
        c_v = idx_v[pl.ds(c * L, L)]
        pos_v = c * L + lid
        for k in range(L):
            plsc.store_scatter(posof_v, [c_v], pos_v, mask=lid == k)


def _make_update(M, D, B):
    spw = M // NW
    assert spw & (spw - 1) == 0
    shift = spw.bit_length() - 1

    def body(mem_hbm, val_hbm, idx_hbm, out_hbm, idx_v, posof_v, dstl_v,
             srcl_v, sidx_v, didx_v, rows_v, semc, sems):
        wid = _worker_id()
        lid = lax.iota(jnp.int32, L)
        rbase = wid * spw

        cdesc = pltpu.async_copy(
            mem_hbm.at[pl.ds(rbase, spw)], out_hbm.at[pl.ds(rbase, spw)], semc
        )

        pltpu.sync_copy(idx_hbm, idx_v)
        _build_posof(idx_v, posof_v, B)

        def cbody(c, n):
            c_v = idx_v[pl.ds(c * L, L)]
            pos_v = c * L + lid
            w_v = plsc.load_gather(posof_v, [c_v])
            mask = (w_v == pos_v) & (
                lax.shift_right_logical(c_v, shift) == wid
            )
            plsc.store_compressed(dstl_v.at[pl.ds(n, L)], c_v, mask=mask)
            plsc.store_compressed(srcl_v.at[pl.ds(n, L)], pos_v, mask=mask)
            return n + plsc.all_reduce_population_count(mask)[0]

        n = lax.fori_loop(0, B // L, cbody, 0)

        @pl.when(n > 0)
        def _():
            d0 = dstl_v[pl.ds(0, L)]
            s0 = srcl_v[pl.ds(0, L)]
            dstl_v[pl.ds(n, L)] = jnp.full((L,), d0[0], jnp.int32)
            srcl_v[pl.ds(n, L)] = jnp.full((L,), s0[0], jnp.int32)

        cdesc.wait()

        @pl.when(n > 0)
        def _():
            nch = lax.div(n + (L - 1), L)

            @pl.loop(0, nch)
            def _(t):
                sidx_v[...] = srcl_v[pl.ds(t * L, L)]
                didx_v[...] = dstl_v[pl.ds(t * L, L)]
                pltpu.async_copy(val_hbm.at[sidx_v], rows_v, sems).wait()
                pltpu.async_copy(rows_v, out_hbm.at[didx_v], sems).wait()

    return pl.kernel(
        body,
        out_type=jax.ShapeDtypeStruct((M, D), jnp.float32),
        mesh=_mesh(),
        scratch_types=[
            pltpu.VMEM((B,), jnp.int32),
            pltpu.VMEM((M,), jnp.int32),
            pltpu.VMEM((B + L,), jnp.int32),
            pltpu.VMEM((B + L,), jnp.int32),
            pltpu.VMEM((L,), jnp.int32),
            pltpu.VMEM((L,), jnp.int32),
            pltpu.VMEM((L, D), jnp.float32),
            pltpu.SemaphoreType.DMA,
            pltpu.SemaphoreType.DMA,
        ],
        compiler_params=_params(),
    )


def _make_retrieve(M, D, B, R):
    rpw = R // NW

    def body(mem_hbm, val_hbm, idx_hbm, ridx_hbm, out_hbm, idx_v, posof_v,
             ridx_v, rows_v, semg):
        wid = _worker_id()
        base = wid * rpw

        pltpu.sync_copy(idx_hbm, idx_v)
        pltpu.sync_copy(ridx_hbm.at[pl.ds(base, rpw)], ridx_v)

        neg1 = jnp.full((L,), -1, jnp.int32)

        @pl.loop(0, M // L)
        def _(i):
            posof_v[pl.ds(i * L, L)] = neg1

        _build_posof(idx_v, posof_v, B)

        for t in range(rpw // L):
            r_v = ridx_v[pl.ds(t * L, L)]
            p_v = plsc.load_gather(posof_v, [r_v])
            pc_v = jnp.maximum(p_v, 0)
            for k in range(L):
                pk, rk, pck = p_v[k], r_v[k], pc_v[k]

                @pl.when(pk >= 0)
                def _(pck=pck, k=k):
                    pltpu.async_copy(
                        val_hbm.at[pl.ds(pck, 1)],
                        rows_v.at[pl.ds(k, 1)], semg,
                    )

                @pl.when(pk < 0)
                def _(rk=rk, k=k):
                    pltpu.async_copy(
                        mem_hbm.at[pl.ds(rk, 1)],
                        rows_v.at[pl.ds(k, 1)], semg,
                    )
            for k in range(L):
                pltpu.make_async_copy(
                    mem_hbm.at[pl.ds(0, 1)], rows_v.at[pl.ds(k, 1)], semg
                ).wait()
            pltpu.sync_copy(rows_v, out_hbm.at[pl.ds(base + t * L, L)])

    return pl.kernel(
        body,
        out_type=jax.ShapeDtypeStruct((R, D), jnp.float32),
        mesh=_mesh(),
        scratch_types=[
            pltpu.VMEM((B,), jnp.int32),
            pltpu.VMEM((M,), jnp.int32),
            pltpu.VMEM((rpw,), jnp.int32),
            pltpu.VMEM((L, D), jnp.float32),
            pltpu.SemaphoreType.DMA,
        ],
        compiler_params=_params(),
    )


def kernel(mem, val, idx, retrieve_idx):
    M, D = mem.shape
    B = idx.shape[0]
    R = retrieve_idx.shape[0]

    new_mem = _make_update(M, D, B)(mem, val, idx)
    retrieved = _make_retrieve(M, D, B, R)(mem, val, idx, retrieve_idx)
    return new_mem, retrieved


# speedup vs baseline: 30.8971x; 30.8971x over previous
import jax
import jax.numpy as jnp
from jax import lax
from jax.experimental import pallas as pl
from jax.experimental.pallas import tpu as pltpu
from jax.experimental.pallas import tpu_sc as plsc

NC, NS, L = 2, 16, 16
NW = NC * NS


def _mesh():
    return plsc.VectorSubcoreMesh(
        core_axis_name="c", subcore_axis_name="s", num_cores=NC, num_subcores=NS
    )


def _params():
    return pltpu.CompilerParams(needs_layout_passes=False)


def _worker_id():
    return lax.axis_index("s") * NC + lax.axis_index("c")


def _build_posof(idx_v, posof_v, B):
    lid = lax.iota(jnp.int32, L)

    @pl.loop(0, B // L)
    def _(c):
        c_v = idx_v[pl.ds(c * L, L)]
        pos_v = c * L + lid
        for k in range(L):
            plsc.store_scatter(posof_v, [c_v], pos_v, mask=lid == k)


def _make_update(M, D, B):
    bpw = B // NW
    nch = bpw // L

    def body(val_hbm, idx_hbm, new_mem_ref, idx_v, posof_v, sidx, didx,
             rows, gsem, ssem):
        wid = _worker_id()
        base = wid * bpw

        pltpu.sync_copy(idx_hbm, idx_v)
        _build_posof(idx_v, posof_v, B)

        def stage(t):
            c_v = idx_v[pl.ds(base + t * L, L)]
            s_v = plsc.load_gather(posof_v, [c_v])
            p = t % 2
            sidx[p][...] = s_v
            didx[p][...] = c_v
            return pltpu.async_copy(val_hbm.at[sidx[p]], rows[p], gsem[p])

        def scatter(t):
            p = t % 2
            return pltpu.async_copy(
                rows[p], new_mem_ref.at[didx[p]], ssem[p]
            )

        gd = [None, None]
        sd = [None, None]
        gd[0] = stage(0)
        for t in range(nch):
            if t + 1 < nch:
                p2 = (t + 1) % 2
                if sd[p2] is not None:
                    sd[p2].wait()
                    sd[p2] = None
                gd[p2] = stage(t + 1)
            gd[t % 2].wait()
            sd[t % 2] = scatter(t)
        for d in sd:
            if d is not None:
                d.wait()

    return pl.kernel(
        body,
        out_type=(),
        mesh=_mesh(),
        scratch_types=[
            pltpu.VMEM((B,), jnp.int32),
            pltpu.VMEM((M,), jnp.int32),
            [pltpu.VMEM((L,), jnp.int32)] * 2,
            [pltpu.VMEM((L,), jnp.int32)] * 2,
            [pltpu.VMEM((L, D), jnp.float32)] * 2,
            [pltpu.SemaphoreType.DMA] * 2,
            [pltpu.SemaphoreType.DMA] * 2,
        ],
        compiler_params=_params(),
    )


def _make_retrieve(M, D, B, R):
    rpw = R // NW

    def body(mem_hbm, val_hbm, idx_hbm, ridx_hbm, out_hbm, idx_v, posof_v,
             ridx_v, rows_v, semg):
        wid = _worker_id()
        base = wid * rpw

        pltpu.sync_copy(idx_hbm, idx_v)
        pltpu.sync_copy(ridx_hbm.at[pl.ds(base, rpw)], ridx_v)

        neg1 = jnp.full((L,), -1, jnp.int32)

        @pl.loop(0, M // L)
        def _(i):
            posof_v[pl.ds(i * L, L)] = neg1

        _build_posof(idx_v, posof_v, B)

        for t in range(rpw // L):
            r_v = ridx_v[pl.ds(t * L, L)]
            p_v = plsc.load_gather(posof_v, [r_v])
            pc_v = jnp.maximum(p_v, 0)
            for k in range(L):
                pk, rk, pck = p_v[k], r_v[k], pc_v[k]

                @pl.when(pk >= 0)
                def _(pck=pck, k=k):
                    pltpu.async_copy(
                        val_hbm.at[pl.ds(pck, 1)],
                        rows_v.at[pl.ds(k, 1)], semg,
                    )

                @pl.when(pk < 0)
                def _(rk=rk, k=k):
                    pltpu.async_copy(
                        mem_hbm.at[pl.ds(rk, 1)],
                        rows_v.at[pl.ds(k, 1)], semg,
                    )
            for k in range(L):
                pltpu.make_async_copy(
                    mem_hbm.at[pl.ds(0, 1)], rows_v.at[pl.ds(k, 1)], semg
                ).wait()
            pltpu.sync_copy(rows_v, out_hbm.at[pl.ds(base + t * L, L)])

    return pl.kernel(
        body,
        out_type=jax.ShapeDtypeStruct((R, D), jnp.float32),
        mesh=_mesh(),
        scratch_types=[
            pltpu.VMEM((B,), jnp.int32),
            pltpu.VMEM((M,), jnp.int32),
            pltpu.VMEM((rpw,), jnp.int32),
            pltpu.VMEM((L, D), jnp.float32),
            pltpu.SemaphoreType.DMA,
        ],
        compiler_params=_params(),
    )


def kernel(mem, val, idx, retrieve_idx):
    M, D = mem.shape
    B = idx.shape[0]
    R = retrieve_idx.shape[0]

    retrieved = _make_retrieve(M, D, B, R)(mem, val, idx, retrieve_idx)
    new_mem_ref = jax.new_ref(mem)
    _make_update(M, D, B)(val, idx, new_mem_ref)
    new_mem = jax.freeze(new_mem_ref)
    return new_mem, retrieved
